# baseline (speedup 1.0000x reference)
import jax
import jax.numpy as jnp
from jax import lax
from jax.experimental import pallas as pl
from jax.experimental.pallas import tpu as pltpu

_NUM_ROWS = 2048
_DIM = 1024
_BATCH = 4
_CH = 2048
_CPB = _NUM_ROWS // _CH
_NC = _BATCH * _CPB
_D = 3


def _body(x_hbm, e_hbm, o_hbm, esum, in_bufs, out_bufs, e_sem, in_sems, out_sems):
    half = _CH // 2

    def in_copies(c, slot):
        b = c // _CPB
        off = (c % _CPB) * _CH
        return [
            pltpu.make_async_copy(
                x_hbm.at[b, pl.ds(off + k * half, half), :],
                in_bufs.at[slot, pl.ds(k * half, half)],
                in_sems.at[slot, k],
            )
            for k in (0, 1)
        ]

    def out_copies(c, slot):
        b = c // _CPB
        off = (c % _CPB) * _CH
        return [
            pltpu.make_async_copy(
                out_bufs.at[slot, pl.ds(k * half, half)],
                o_hbm.at[b, pl.ds(off + k * half, half), :],
                out_sems.at[slot, k],
            )
            for k in (0, 1)
        ]

    e_cp = pltpu.make_async_copy(e_hbm.at[0], esum, e_sem)
    e_cp.start()
    for k in range(_D):
        for cp in in_copies(k, k):
            cp.start()

    e_cp.wait()
    e0 = esum[0:8]
    perm_head = jnp.concatenate(
        [e0[0:1], e0[3:4], e0[6:7], e0[1:2], e0[4:5], e0[7:8], e0[2:3], e0[5:6]],
        axis=0,
    )
    esum[0:8] = e0 + perm_head
    esum[8:] = 2.0 * esum[8:]

    def loop_body(c, _):
        slot = c % _D
        for cp in in_copies(c, slot):
            cp.wait()

        @pl.when(c >= _D)
        def _():
            for cp in out_copies(c - _D, slot):
                cp.wait()

        off = (c % _CPB) * _CH
        out_bufs[slot] = in_bufs[slot] + esum[pl.ds(off, _CH)]

        @pl.when(c + _D < _NC)
        def _():
            for cp in in_copies(c + _D, slot):
                cp.start()

        for cp in out_copies(c, slot):
            cp.start()
        return 0

    lax.fori_loop(0, _NC, loop_body, 0)

    def drain(c, _):
        for cp in out_copies(c, c % _D):
            cp.wait()
        return 0

    lax.fori_loop(_NC - _D, _NC, drain, 0)


def kernel(x, embedding):
    return pl.pallas_call(
        _body,
        in_specs=[
            pl.BlockSpec(memory_space=pltpu.MemorySpace.HBM),
            pl.BlockSpec(memory_space=pltpu.MemorySpace.HBM),
        ],
        out_specs=pl.BlockSpec(memory_space=pltpu.MemorySpace.HBM),
        out_shape=jax.ShapeDtypeStruct(x.shape, x.dtype),
        scratch_shapes=[
            pltpu.VMEM((_NUM_ROWS, _DIM), jnp.float32),
            pltpu.VMEM((_D, _CH, _DIM), jnp.float32),
            pltpu.VMEM((_D, _CH, _DIM), jnp.float32),
            pltpu.SemaphoreType.DMA,
            pltpu.SemaphoreType.DMA((_D, 2)),
            pltpu.SemaphoreType.DMA((_D, 2)),
        ],
    )(x, embedding)


# speedup vs baseline: 1.0494x; 1.0494x over previous
import jax
import jax.numpy as jnp
from jax import lax
from jax.experimental import pallas as pl
from jax.experimental.pallas import tpu as pltpu

_NUM_ROWS = 2048
_DIM = 1024
_BATCH = 4
_CH = 2048
_CPB = _NUM_ROWS // _CH
_NC = _BATCH * _CPB
_D = 3


def _body(x_hbm, e_hbm, o_hbm, esum, in_bufs, out_bufs, e_sem, in_sems, out_sems):
    def in_copy(c, slot):
        b = c // _CPB
        off = (c % _CPB) * _CH
        return pltpu.make_async_copy(
            x_hbm.at[b, pl.ds(off, _CH), :],
            in_bufs.at[slot],
            in_sems.at[slot],
        )

    def out_copy(c, slot):
        b = c // _CPB
        off = (c % _CPB) * _CH
        return pltpu.make_async_copy(
            out_bufs.at[slot],
            o_hbm.at[b, pl.ds(off, _CH), :],
            out_sems.at[slot],
        )

    e_cp = pltpu.make_async_copy(e_hbm.at[0], esum, e_sem)
    e_cp.start()
    for k in range(_D):
        in_copy(k, k).start()

    e_cp.wait()
    e0 = esum[0:8]
    perm_head = jnp.concatenate(
        [e0[0:1], e0[3:4], e0[6:7], e0[1:2], e0[4:5], e0[7:8], e0[2:3], e0[5:6]],
        axis=0,
    )
    esum[0:8] = e0 + perm_head
    esum[8:] = 2.0 * esum[8:]

    def loop_body(c, _):
        slot = c % _D
        in_copy(c, slot).wait()

        @pl.when(c >= _D)
        def _():
            out_copy(c - _D, slot).wait()

        off = (c % _CPB) * _CH
        out_bufs[slot] = in_bufs[slot] + esum[pl.ds(off, _CH)]

        @pl.when(c + _D < _NC)
        def _():
            in_copy(c + _D, slot).start()

        out_copy(c, slot).start()
        return 0

    lax.fori_loop(0, _NC, loop_body, 0)

    def drain(c, _):
        out_copy(c, c % _D).wait()
        return 0

    lax.fori_loop(_NC - _D, _NC, drain, 0)


def kernel(x, embedding):
    return pl.pallas_call(
        _body,
        in_specs=[
            pl.BlockSpec(memory_space=pltpu.MemorySpace.HBM),
            pl.BlockSpec(memory_space=pltpu.MemorySpace.HBM),
        ],
        out_specs=pl.BlockSpec(memory_space=pltpu.MemorySpace.HBM),
        out_shape=jax.ShapeDtypeStruct(x.shape, x.dtype),
        scratch_shapes=[
            pltpu.VMEM((_NUM_ROWS, _DIM), jnp.float32),
            pltpu.VMEM((_D, _CH, _DIM), jnp.float32),
            pltpu.VMEM((_D, _CH, _DIM), jnp.float32),
            pltpu.SemaphoreType.DMA,
            pltpu.SemaphoreType.DMA((_D,)),
            pltpu.SemaphoreType.DMA((_D,)),
        ],
    )(x, embedding)
